# baseline (speedup 1.0000x reference)
import functools

import jax
import jax.numpy as jnp
from jax import lax
from jax.experimental import pallas as pl
from jax.experimental.pallas import tpu as pltpu
from jax.experimental.pallas import tpu_sc as plsc

D_MODEL = 64
EPS = 1e-05

NUM_CORES = 2
NUM_SUBCORES = 16
NUM_WORKERS = NUM_CORES * NUM_SUBCORES

BBLK = 128
TBLK = 8


def _sc_embed_logmap_t(tok_t, emb_packed, seq, bsz):
    n_bblk = bsz // BBLK
    n_tgrp = seq // TBLK
    blocks = n_tgrp * n_bblk
    per_w = blocks // NUM_WORKERS
    mesh = plsc.VectorSubcoreMesh(core_axis_name="c", subcore_axis_name="s")

    @functools.partial(
        pl.kernel,
        mesh=mesh,
        out_type=jax.ShapeDtypeStruct((seq, D_MODEL, bsz), jnp.float32),
        scratch_types=[
            pltpu.VMEM((TBLK, BBLK), jnp.int32),
            pltpu.VMEM((TBLK, BBLK), jnp.int32),
            pltpu.VMEM((2, BBLK, 128), jnp.float32),
            pltpu.VMEM((2, D_MODEL, BBLK), jnp.float32),
            pltpu.VMEM((BBLK,), jnp.float32),
            pltpu.SemaphoreType.DMA,
            pltpu.SemaphoreType.DMA,
            pltpu.SemaphoreType.DMA,
        ],
        compiler_params=pltpu.CompilerParams(needs_layout_passes=False),
    )
    def body(tok_hbm, tab_hbm, out_hbm, idx_v, k_v, rows_v, obuf_v, dv_v,
             gsem, osem, isem):
        wid = lax.axis_index("s") * NUM_CORES + lax.axis_index("c")
        lanes = lax.iota(jnp.int32, 16)

        def superblock(sb, carry):
            blk = wid * per_w + sb
            tg = blk // n_bblk
            bb = blk % n_bblk
            t0 = pl.multiple_of(tg * TBLK, TBLK)
            b0 = pl.multiple_of(bb * BBLK, BBLK)
            pltpu.sync_copy(
                tok_hbm.at[pl.ds(t0, TBLK), pl.ds(b0, BBLK)], idx_v)
            def mk_k(j, c):
                def mk16(q, c2):
                    v = idx_v[j, pl.ds(q * 16, 16)]
                    k_v[j, pl.ds(q * 16, 16)] = lax.shift_right_logical(v, 1)
                    return c2
                return lax.fori_loop(0, BBLK // 16, mk16, c)
            lax.fori_loop(0, TBLK, mk_k, 0)

            def t_step(j, carry2):
                buf = j % 2
                pltpu.async_copy(
                    tab_hbm.at[k_v.at[j]], rows_v.at[buf], gsem).wait()

                def grp(q, c3):
                    v = idx_v[j, pl.ds(q * 16, 16)]
                    dv = lax.mul(lax.bitwise_and(v, 1), 64)
                    row_idx = q * 16 + lanes
                    acc = jnp.zeros((16,), jnp.float32)

                    def norm_c(c, a):
                        x = plsc.load_gather(
                            rows_v.at[buf], [row_idx, dv + c])
                        return a + x * x
                    acc = lax.fori_loop(0, D_MODEL, norm_c, acc, unroll=8)
                    scale = 2.0 / ((1.0 + EPS) - acc)

                    def write_c(c, c4):
                        x = plsc.load_gather(
                            rows_v.at[buf], [row_idx, dv + c])
                        obuf_v[buf, c, pl.ds(q * 16, 16)] = x * scale
                        return c4
                    return lax.fori_loop(0, D_MODEL, write_c, c3, unroll=8)
                lax.fori_loop(0, TBLK, grp, 0)

                pltpu.async_copy(
                    obuf_v.at[buf],
                    out_hbm.at[t0 + j, :, pl.ds(b0, BBLK)],
                    osem).wait()
                return carry2

            lax.fori_loop(0, TBLK, t_step, 0)
            return carry

        lax.fori_loop(0, per_w, superblock, 0)

    return body(tok_t, emb_packed)


def kernel(token_ids, embeddings):
    bsz, seq = token_ids.shape
    vocab, d = embeddings.shape
    tok_t = token_ids.T.astype(jnp.int32)
    emb_packed = embeddings.reshape(vocab // 2, 2 * d)
    out_t = _sc_embed_logmap_t(tok_t, emb_packed, seq, bsz)
    return out_t.transpose(2, 0, 1)


# speedup vs baseline: 1.0658x; 1.0658x over previous
import functools

import jax
import jax.numpy as jnp
from jax import lax
from jax.experimental import pallas as pl
from jax.experimental.pallas import tpu as pltpu
from jax.experimental.pallas import tpu_sc as plsc

D_MODEL = 64
EPS = 1e-05

NUM_CORES = 2
NUM_SUBCORES = 16
NUM_WORKERS = NUM_CORES * NUM_SUBCORES

BBLK = 128
TBLK = 8
GBATCH = 4


def _sc_embed_logmap_t(tok_t, emb_packed, seq, bsz):
    n_bblk = bsz // BBLK
    n_tgrp = seq // TBLK
    blocks = n_tgrp * n_bblk
    per_w = blocks // NUM_WORKERS
    mesh = plsc.VectorSubcoreMesh(core_axis_name="c", subcore_axis_name="s")

    @functools.partial(
        pl.kernel,
        mesh=mesh,
        out_type=jax.ShapeDtypeStruct((seq, D_MODEL, bsz), jnp.float32),
        scratch_types=[
            pltpu.VMEM((TBLK, BBLK), jnp.int32),
            pltpu.VMEM((TBLK, BBLK), jnp.int32),
            pltpu.VMEM((GBATCH, BBLK, 128), jnp.float32),
            pltpu.VMEM((2, D_MODEL, BBLK), jnp.float32),
            pltpu.SemaphoreType.DMA,
            pltpu.SemaphoreType.DMA,
        ],
        compiler_params=pltpu.CompilerParams(needs_layout_passes=False),
    )
    def body(tok_hbm, tab_hbm, out_hbm, idx_v, k_v, rows_v, obuf_v,
             gsem, osem):
        wid = lax.axis_index("s") * NUM_CORES + lax.axis_index("c")
        lanes = lax.iota(jnp.int32, 16)

        def superblock(sb, carry):
            blk = wid * per_w + sb
            tg = blk // n_bblk
            bb = blk % n_bblk
            t0 = pl.multiple_of(tg * TBLK, TBLK)
            b0 = pl.multiple_of(bb * BBLK, BBLK)
            pltpu.sync_copy(
                tok_hbm.at[pl.ds(t0, TBLK), pl.ds(b0, BBLK)], idx_v)

            def mk_k(j, c):
                def mk16(q, c2):
                    v = idx_v[j, pl.ds(q * 16, 16)]
                    k_v[j, pl.ds(q * 16, 16)] = lax.shift_right_logical(v, 1)
                    return c2
                return lax.fori_loop(0, BBLK // 16, mk16, c)
            lax.fori_loop(0, TBLK, mk_k, 0)

            def wave(w, carry1):
                jbase = w * GBATCH
                for g in range(GBATCH):
                    pltpu.async_copy(
                        tab_hbm.at[k_v.at[jbase + g]], rows_v.at[g], gsem)
                for g in range(GBATCH):
                    pltpu.make_async_copy(
                        tab_hbm.at[k_v.at[jbase + g]], rows_v.at[g],
                        gsem).wait()

                def t_step(g, carry2):
                    j = jbase + g
                    buf = j % 2

                    def grp(q, c3):
                        v = idx_v[j, pl.ds(q * 16, 16)]
                        dv = lax.mul(lax.bitwise_and(v, 1), 64)
                        row_idx = q * 16 + lanes
                        acc = jnp.zeros((16,), jnp.float32)

                        def norm_c(c, a):
                            x = plsc.load_gather(
                                rows_v.at[g], [row_idx, dv + c])
                            return a + x * x
                        acc = lax.fori_loop(0, D_MODEL, norm_c, acc,
                                            unroll=16)
                        scale = 2.0 / ((1.0 + EPS) - acc)

                        def write_c(c, c4):
                            x = plsc.load_gather(
                                rows_v.at[g], [row_idx, dv + c])
                            obuf_v[buf, c, pl.ds(q * 16, 16)] = x * scale
                            return c4
                        return lax.fori_loop(0, D_MODEL, write_c, c3,
                                             unroll=16)
                    lax.fori_loop(0, BBLK // 16, grp, 0)

                    @pl.when(j >= 2)
                    def _():
                        pltpu.make_async_copy(
                            obuf_v.at[buf],
                            out_hbm.at[t0 + j - 2, :, pl.ds(b0, BBLK)],
                            osem).wait()
                    pltpu.async_copy(
                        obuf_v.at[buf],
                        out_hbm.at[t0 + j, :, pl.ds(b0, BBLK)],
                        osem)
                    return carry2

                lax.fori_loop(0, GBATCH, t_step, 0)
                return carry1

            lax.fori_loop(0, TBLK // GBATCH, wave, 0)

            for j in (TBLK - 2, TBLK - 1):
                pltpu.make_async_copy(
                    obuf_v.at[j % 2],
                    out_hbm.at[t0 + j, :, pl.ds(b0, BBLK)],
                    osem).wait()
            return carry

        lax.fori_loop(0, per_w, superblock, 0)

    return body(tok_t, emb_packed)


def kernel(token_ids, embeddings):
    bsz, seq = token_ids.shape
    vocab, d = embeddings.shape
    tok_t = token_ids.T.astype(jnp.int32)
    emb_packed = embeddings.reshape(vocab // 2, 2 * d)
    out_t = _sc_embed_logmap_t(tok_t, emb_packed, seq, bsz)
    return out_t.transpose(2, 0, 1)


# speedup vs baseline: 1.4627x; 1.3725x over previous
import functools

import jax
import jax.numpy as jnp
from jax import lax
from jax.experimental import pallas as pl
from jax.experimental.pallas import tpu as pltpu
from jax.experimental.pallas import tpu_sc as plsc

D_MODEL = 64
EPS = 1e-05

NUM_CORES = 2
NUM_SUBCORES = 16
NUM_WORKERS = NUM_CORES * NUM_SUBCORES

_DN = lax.GatherDimensionNumbers(
    offset_dims=(), collapsed_slice_dims=(0,), start_index_map=(0,))


def _lane_perm(x, idx):
    return lax.gather(x, idx[:, None], _DN, slice_sizes=(1,),
                      mode=lax.GatherScatterMode.PROMISE_IN_BOUNDS)


def _lane_allsum(x):
    lanes = lax.iota(jnp.int32, 16)
    for sh in (8, 4, 2, 1):
        x = x + _lane_perm(x, lanes ^ sh)
    return x


def _transpose16(regs, lanes):
    out = list(regs)
    s = 1
    while s < 16:
        nxt = list(out)
        m = (lanes & s) != 0
        down = (lanes - s) & 15
        up = (lanes + s) & 15
        for i in range(16):
            if i & s:
                continue
            a, b = out[i], out[i + s]
            nxt[i] = jnp.where(m, _lane_perm(b, down), a)
            nxt[i + s] = jnp.where(m, b, _lane_perm(a, up))
        out = nxt
        s *= 2
    return out


def _repack_table(emb_t, tail_emb, vocab):
    n_blk = vocab // 128 + (1 if vocab % 128 else 0)
    per_w = (n_blk + NUM_WORKERS - 1) // NUM_WORKERS
    mesh = plsc.VectorSubcoreMesh(core_axis_name="c", subcore_axis_name="s")

    @functools.partial(
        pl.kernel,
        mesh=mesh,
        out_type=jax.ShapeDtypeStruct((vocab, 128), jnp.float32),
        scratch_types=[
            pltpu.VMEM((2, D_MODEL, 128), jnp.float32),
            pltpu.VMEM((2, 128, 128), jnp.float32),
            pltpu.VMEM((D_MODEL, 64), jnp.float32),
            pltpu.SemaphoreType.DMA,
            pltpu.SemaphoreType.DMA,
        ],
        compiler_params=pltpu.CompilerParams(needs_layout_passes=False),
    )
    def body(emb_hbm, tail_hbm, out_hbm, in_v, tr_v, tail_v, isem, osem):
        wid = lax.axis_index("s") * NUM_CORES + lax.axis_index("c")
        lanes = lax.iota(jnp.int32, 16)

        def transpose_block(buf, n_vt):
            def vt_step(vt, c1):
                for ct in range(4):
                    regs = [in_v[buf, ct * 16 + i, pl.ds(vt * 16, 16)]
                            for i in range(16)]
                    tr = _transpose16(regs, lanes)
                    for i in range(16):
                        tr_v[buf, vt * 16 + i, pl.ds(ct * 16, 16)] = tr[i]
                return c1
            lax.fori_loop(0, n_vt, vt_step, 0)

        def step(t, carry):
            blk = wid + t * NUM_WORKERS
            buf = t % 2
            v0 = pl.multiple_of(blk * 128, 128)

            @pl.when(blk < n_blk - 1)
            def _():
                pltpu.sync_copy(
                    emb_hbm.at[:, pl.ds(v0, 128)], in_v.at[buf])
                transpose_block(buf, 8)
                pltpu.sync_copy(
                    tr_v.at[buf], out_hbm.at[pl.ds(v0, 128)])

            @pl.when(blk == n_blk - 1)
            def _():
                pltpu.sync_copy(tail_hbm, tail_v)

                def tail_row(i, c1):
                    for k in range(4):
                        tr_v[buf, i, pl.ds(16 * k, 16)] = (
                            tail_v[i, pl.ds(16 * k, 16)])
                    return c1
                lax.fori_loop(0, 64, tail_row, 0)
                pltpu.sync_copy(
                    tr_v.at[buf, pl.ds(0, 64)],
                    out_hbm.at[pl.ds(v0, 64)])
            return carry

        lax.fori_loop(0, per_w, step, 0)

    return body(emb_t, tail_emb)


CHUNK2 = 256


def _gather_logmap(idx2d, pad_tab, n_rows):
    per_w = n_rows // NUM_WORKERS
    steps = per_w // CHUNK2
    mesh = plsc.VectorSubcoreMesh(core_axis_name="c", subcore_axis_name="s")

    @functools.partial(
        pl.kernel,
        mesh=mesh,
        out_type=jax.ShapeDtypeStruct((n_rows, D_MODEL), jnp.float32),
        scratch_types=[
            pltpu.VMEM((CHUNK2 // 128, 128), jnp.int32),
            pltpu.VMEM((CHUNK2, 128), jnp.float32),
            pltpu.VMEM((CHUNK2, D_MODEL), jnp.float32),
            pltpu.SemaphoreType.DMA,
            pltpu.SemaphoreType.DMA,
        ],
        compiler_params=pltpu.CompilerParams(needs_layout_passes=False),
    )
    def body(idx_hbm, tab_hbm, out_hbm, idx_v, rows_v, obuf_v, gsem, osem):
        wid = lax.axis_index("s") * NUM_CORES + lax.axis_index("c")
        row_base = wid * per_w
        irow_base = row_base // 128

        def step(g, carry):
            row_off = pl.multiple_of(row_base + g * CHUNK2, CHUNK2)
            idx_off = pl.multiple_of(
                irow_base + g * (CHUNK2 // 128), CHUNK2 // 128)
            pltpu.sync_copy(
                idx_hbm.at[pl.ds(idx_off, CHUNK2 // 128)], idx_v)
            copies = [
                pltpu.async_copy(
                    tab_hbm.at[idx_v.at[b]],
                    rows_v.at[pl.ds(b * 128, 128)], gsem)
                for b in range(CHUNK2 // 128)
            ]
            for c in copies:
                c.wait()

            def row_fix(i, c):
                q = [rows_v[i, pl.ds(16 * k, 16)] for k in range(4)]
                s = (q[0] * q[0] + q[1] * q[1]) + (q[2] * q[2] + q[3] * q[3])
                nsv = _lane_allsum(s)
                scale = 2.0 / ((1.0 + EPS) - nsv)
                for k in range(4):
                    obuf_v[i, pl.ds(16 * k, 16)] = q[k] * scale
                return c

            lax.fori_loop(0, CHUNK2, row_fix, 0, unroll=4)
            pltpu.sync_copy(obuf_v, out_hbm.at[pl.ds(row_off, CHUNK2)])
            return carry

        lax.fori_loop(0, steps, step, 0)

    return body(idx2d, pad_tab)


def kernel(token_ids, embeddings):
    bsz, seq = token_ids.shape
    vocab, d = embeddings.shape
    n_rows = bsz * seq
    n_full = (vocab // 128) * 128
    pad_tab = _repack_table(embeddings.T, embeddings[n_full:, :], vocab)
    idx2d = token_ids.reshape(n_rows // 128, 128).astype(jnp.int32)
    out = _gather_logmap(idx2d, pad_tab, n_rows)
    return out.reshape(bsz, seq, d)


# speedup vs baseline: 2.4016x; 1.6419x over previous
import functools

import jax
import jax.numpy as jnp
from jax import lax
from jax.experimental import pallas as pl
from jax.experimental.pallas import tpu as pltpu
from jax.experimental.pallas import tpu_sc as plsc

D_MODEL = 64
EPS = 1e-05

NUM_CORES = 2
NUM_SUBCORES = 16
NUM_WORKERS = NUM_CORES * NUM_SUBCORES

_DN = lax.GatherDimensionNumbers(
    offset_dims=(), collapsed_slice_dims=(0,), start_index_map=(0,))


def _lane_perm(x, idx):
    return lax.gather(x, idx[:, None], _DN, slice_sizes=(1,),
                      mode=lax.GatherScatterMode.PROMISE_IN_BOUNDS)


def _lane_allsum(x):
    lanes = lax.iota(jnp.int32, 16)
    for sh in (8, 4, 2, 1):
        x = x + _lane_perm(x, lanes ^ sh)
    return x


def _transpose16(regs, lanes):
    out = list(regs)
    s = 1
    while s < 16:
        nxt = list(out)
        m = (lanes & s) != 0
        down = (lanes - s) & 15
        up = (lanes + s) & 15
        for i in range(16):
            if i & s:
                continue
            a, b = out[i], out[i + s]
            nxt[i] = jnp.where(m, _lane_perm(b, down), a)
            nxt[i + s] = jnp.where(m, b, _lane_perm(a, up))
        out = nxt
        s *= 2
    return out


def _repack_table(emb_t, tail_emb, vocab):
    n_blk = vocab // 128 + (1 if vocab % 128 else 0)
    n_main = n_blk - 1
    per_w = (n_blk + NUM_WORKERS - 1) // NUM_WORKERS
    mesh = plsc.VectorSubcoreMesh(core_axis_name="c", subcore_axis_name="s")

    @functools.partial(
        pl.kernel,
        mesh=mesh,
        out_type=jax.ShapeDtypeStruct((vocab, 128), jnp.float32),
        scratch_types=[
            pltpu.VMEM((2, D_MODEL, 128), jnp.float32),
            pltpu.VMEM((2, 128, 128), jnp.float32),
            pltpu.VMEM((D_MODEL, 64), jnp.float32),
            pltpu.VMEM((D_MODEL, 128), jnp.float32),
            pltpu.SemaphoreType.DMA,
            pltpu.SemaphoreType.DMA,
        ],
        compiler_params=pltpu.CompilerParams(needs_layout_passes=False),
    )
    def body(emb_hbm, tail_hbm, out_hbm, in_v, tr_v, tail_v, tail_o,
             isem, osem):
        wid = lax.axis_index("s") * NUM_CORES + lax.axis_index("c")
        lanes = lax.iota(jnp.int32, 16)

        def in_copy(blk, buf):
            v0 = pl.multiple_of(blk * 128, 128)
            return pltpu.make_async_copy(
                emb_hbm.at[:, pl.ds(v0, 128)], in_v.at[buf], isem)

        def out_copy(blk, buf):
            v0 = pl.multiple_of(blk * 128, 128)
            return pltpu.make_async_copy(
                tr_v.at[buf], out_hbm.at[pl.ds(v0, 128)], osem)

        def transpose_block(buf, n_vt):
            def vt_step(vt, c1):
                for ct in range(4):
                    regs = [in_v[buf, ct * 16 + i, pl.ds(vt * 16, 16)]
                            for i in range(16)]
                    tr = _transpose16(regs, lanes)
                    for i in range(16):
                        tr_v[buf, vt * 16 + i, pl.ds(ct * 16, 16)] = tr[i]
                return c1
            lax.fori_loop(0, n_vt, vt_step, 0)

        @pl.when(wid < n_main)
        def _():
            in_copy(wid, 0).start()

        def step(t, carry):
            blk = wid + t * NUM_WORKERS
            buf = t % 2

            @pl.when(blk < n_main)
            def _():
                nxt = blk + NUM_WORKERS

                @pl.when(nxt < n_main)
                def _():
                    in_copy(nxt, (t + 1) % 2).start()
                in_copy(blk, buf).wait()

                @pl.when(t >= 2)
                def _():
                    out_copy(blk - 2 * NUM_WORKERS, buf).wait()
                transpose_block(buf, 8)
                out_copy(blk, buf).start()

            @pl.when(blk == n_blk - 1)
            def _():
                v0 = pl.multiple_of((n_blk - 1) * 128, 128)
                pltpu.sync_copy(tail_hbm, tail_v)

                def tail_row(i, c1):
                    for k in range(4):
                        tail_o[i, pl.ds(16 * k, 16)] = (
                            tail_v[i, pl.ds(16 * k, 16)])
                    return c1
                lax.fori_loop(0, 64, tail_row, 0)
                pltpu.sync_copy(tail_o, out_hbm.at[pl.ds(v0, 64)])
            return carry

        lax.fori_loop(0, per_w, step, 0)

        n_mine = (n_main - wid + NUM_WORKERS - 1) // NUM_WORKERS

        @pl.when(n_mine >= 1)
        def _():
            t_last = n_mine - 1
            out_copy(wid + t_last * NUM_WORKERS, t_last % 2).wait()

        @pl.when(n_mine >= 2)
        def _():
            t_prev = n_mine - 2
            out_copy(wid + t_prev * NUM_WORKERS, t_prev % 2).wait()

    return body(emb_t, tail_emb)


CHUNK2 = 128
NG = CHUNK2 // 128


def _gather_logmap(idx2d, pad_tab, n_rows):
    per_w = n_rows // NUM_WORKERS
    steps = per_w // CHUNK2
    mesh = plsc.VectorSubcoreMesh(core_axis_name="c", subcore_axis_name="s")

    @functools.partial(
        pl.kernel,
        mesh=mesh,
        out_type=jax.ShapeDtypeStruct((n_rows, D_MODEL), jnp.float32),
        scratch_types=[
            pltpu.VMEM((2, NG, 128), jnp.int32),
            pltpu.VMEM((2, CHUNK2, 128), jnp.float32),
            pltpu.VMEM((2, CHUNK2, D_MODEL), jnp.float32),
            pltpu.SemaphoreType.DMA,
            pltpu.SemaphoreType.DMA,
            pltpu.SemaphoreType.DMA,
        ],
        compiler_params=pltpu.CompilerParams(needs_layout_passes=False),
    )
    def body(idx_hbm, tab_hbm, out_hbm, idx_v, rows_v, obuf_v,
             isem, gsem, osem):
        wid = lax.axis_index("s") * NUM_CORES + lax.axis_index("c")
        row_base = wid * per_w
        irow_base = row_base // 128

        def idx_copy(g, buf):
            off = pl.multiple_of(irow_base + g * NG, NG)
            return pltpu.make_async_copy(
                idx_hbm.at[pl.ds(off, NG)], idx_v.at[buf], isem)

        def gather_copy(g, buf, b):
            return pltpu.make_async_copy(
                tab_hbm.at[idx_v.at[buf, b]],
                rows_v.at[buf, pl.ds(b * 128, 128)], gsem)

        def gather_wait(buf, b):
            return pltpu.make_async_copy(
                tab_hbm.at[pl.ds(0, 128)],
                rows_v.at[buf, pl.ds(b * 128, 128)], gsem)

        def out_copy(g, buf):
            off = pl.multiple_of(row_base + g * CHUNK2, CHUNK2)
            return pltpu.make_async_copy(
                obuf_v.at[buf], out_hbm.at[pl.ds(off, CHUNK2)], osem)

        idx_copy(0, 0).start()
        idx_copy(0, 0).wait()
        for b in range(NG):
            gather_copy(0, 0, b).start()
        idx_copy(1, 1).start()

        def step(g, carry):
            buf = g % 2

            @pl.when(g + 1 < steps)
            def _():
                idx_copy(g + 1, (g + 1) % 2).wait()
                for b in range(NG):
                    gather_copy(g + 1, (g + 1) % 2, b).start()

            for b in range(NG):
                gather_wait(buf, b).wait()

            @pl.when(g + 2 < steps)
            def _():
                idx_copy(g + 2, g % 2).start()

            def row_fix(i, c):
                q = [rows_v[buf, i, pl.ds(16 * k, 16)] for k in range(4)]
                s = (q[0] * q[0] + q[1] * q[1]) + (q[2] * q[2] + q[3] * q[3])
                nsv = _lane_allsum(s)
                scale = 2.0 / ((1.0 + EPS) - nsv)
                for k in range(4):
                    obuf_v[buf, i, pl.ds(16 * k, 16)] = q[k] * scale
                return c

            lax.fori_loop(0, CHUNK2, row_fix, 0, unroll=8)

            @pl.when(g >= 2)
            def _():
                out_copy(g - 2, buf).wait()
            out_copy(g, buf).start()
            return carry

        lax.fori_loop(0, steps, step, 0)
        out_copy(steps - 2, (steps - 2) % 2).wait()
        out_copy(steps - 1, (steps - 1) % 2).wait()

    return body(idx2d, pad_tab)


def kernel(token_ids, embeddings):
    bsz, seq = token_ids.shape
    vocab, d = embeddings.shape
    n_rows = bsz * seq
    n_full = (vocab // 128) * 128
    pad_tab = _repack_table(embeddings.T, embeddings[n_full:, :], vocab)
    idx2d = token_ids.reshape(n_rows // 128, 128).astype(jnp.int32)
    out = _gather_logmap(idx2d, pad_tab, n_rows)
    return out.reshape(bsz, seq, d)


# speedup vs baseline: 2.4052x; 1.0015x over previous
import functools

import jax
import jax.numpy as jnp
from jax import lax
from jax.experimental import pallas as pl
from jax.experimental.pallas import tpu as pltpu
from jax.experimental.pallas import tpu_sc as plsc

D_MODEL = 64
EPS = 1e-05

NUM_CORES = 2
NUM_SUBCORES = 16
NUM_WORKERS = NUM_CORES * NUM_SUBCORES

_DN = lax.GatherDimensionNumbers(
    offset_dims=(), collapsed_slice_dims=(0,), start_index_map=(0,))


def _lane_perm(x, idx):
    return lax.gather(x, idx[:, None], _DN, slice_sizes=(1,),
                      mode=lax.GatherScatterMode.PROMISE_IN_BOUNDS)


def _lane_allsum(x):
    lanes = lax.iota(jnp.int32, 16)
    for sh in (8, 4, 2, 1):
        x = x + _lane_perm(x, lanes ^ sh)
    return x


def _transpose16(regs, lanes):
    out = list(regs)
    s = 1
    while s < 16:
        nxt = list(out)
        m = (lanes & s) != 0
        down = (lanes - s) & 15
        up = (lanes + s) & 15
        for i in range(16):
            if i & s:
                continue
            a, b = out[i], out[i + s]
            nxt[i] = jnp.where(m, _lane_perm(b, down), a)
            nxt[i + s] = jnp.where(m, b, _lane_perm(a, up))
        out = nxt
        s *= 2
    return out


def _repack_table(emb_t, tail_emb, vocab):
    n_blk = vocab // 128 + (1 if vocab % 128 else 0)
    n_main = n_blk - 1
    per_w = (n_blk + NUM_WORKERS - 1) // NUM_WORKERS
    mesh = plsc.VectorSubcoreMesh(core_axis_name="c", subcore_axis_name="s")

    @functools.partial(
        pl.kernel,
        mesh=mesh,
        out_type=jax.ShapeDtypeStruct((vocab, 128), jnp.float32),
        scratch_types=[
            pltpu.VMEM((2, D_MODEL, 128), jnp.float32),
            pltpu.VMEM((2, 128, 128), jnp.float32),
            pltpu.VMEM((D_MODEL, 64), jnp.float32),
            pltpu.VMEM((D_MODEL, 128), jnp.float32),
            pltpu.SemaphoreType.DMA,
            pltpu.SemaphoreType.DMA,
        ],
        compiler_params=pltpu.CompilerParams(needs_layout_passes=False),
    )
    def body(emb_hbm, tail_hbm, out_hbm, in_v, tr_v, tail_v, tail_o,
             isem, osem):
        wid = lax.axis_index("s") * NUM_CORES + lax.axis_index("c")
        lanes = lax.iota(jnp.int32, 16)

        def in_copy(blk, buf):
            v0 = pl.multiple_of(blk * 128, 128)
            return pltpu.make_async_copy(
                emb_hbm.at[:, pl.ds(v0, 128)], in_v.at[buf], isem)

        def out_copy(blk, buf):
            v0 = pl.multiple_of(blk * 128, 128)
            return pltpu.make_async_copy(
                tr_v.at[buf], out_hbm.at[pl.ds(v0, 128)], osem)

        def transpose_block(buf, n_vt):
            def vt_step(vt, c1):
                for ct in range(4):
                    regs = [in_v[buf, ct * 16 + i, pl.ds(vt * 16, 16)]
                            for i in range(16)]
                    tr = _transpose16(regs, lanes)
                    for i in range(16):
                        tr_v[buf, vt * 16 + i, pl.ds(ct * 16, 16)] = tr[i]
                return c1
            lax.fori_loop(0, n_vt, vt_step, 0)

        @pl.when(wid < n_main)
        def _():
            in_copy(wid, 0).start()

        def step(t, carry):
            blk = wid + t * NUM_WORKERS
            buf = t % 2

            @pl.when(blk < n_main)
            def _():
                nxt = blk + NUM_WORKERS

                @pl.when(nxt < n_main)
                def _():
                    in_copy(nxt, (t + 1) % 2).start()
                in_copy(blk, buf).wait()

                @pl.when(t >= 2)
                def _():
                    out_copy(blk - 2 * NUM_WORKERS, buf).wait()
                transpose_block(buf, 8)
                out_copy(blk, buf).start()

            @pl.when(blk == n_blk - 1)
            def _():
                v0 = pl.multiple_of((n_blk - 1) * 128, 128)
                pltpu.sync_copy(tail_hbm, tail_v)

                def tail_row(i, c1):
                    for k in range(4):
                        tail_o[i, pl.ds(16 * k, 16)] = (
                            tail_v[i, pl.ds(16 * k, 16)])
                    return c1
                lax.fori_loop(0, 64, tail_row, 0)
                pltpu.sync_copy(tail_o, out_hbm.at[pl.ds(v0, 64)])
            return carry

        lax.fori_loop(0, per_w, step, 0)

        n_mine = (n_main - wid + NUM_WORKERS - 1) // NUM_WORKERS

        @pl.when(n_mine >= 1)
        def _():
            t_last = n_mine - 1
            out_copy(wid + t_last * NUM_WORKERS, t_last % 2).wait()

        @pl.when(n_mine >= 2)
        def _():
            t_prev = n_mine - 2
            out_copy(wid + t_prev * NUM_WORKERS, t_prev % 2).wait()

    return body(emb_t, tail_emb)


CHUNK2 = 128
NG = CHUNK2 // 128


def _gather_logmap(idx2d, pad_tab, n_rows):
    per_w = n_rows // NUM_WORKERS
    steps = per_w // CHUNK2
    mesh = plsc.VectorSubcoreMesh(core_axis_name="c", subcore_axis_name="s")

    @functools.partial(
        pl.kernel,
        mesh=mesh,
        out_type=jax.ShapeDtypeStruct((n_rows, D_MODEL), jnp.float32),
        scratch_types=[
            pltpu.VMEM((4, NG, 128), jnp.int32),
            pltpu.VMEM((4, CHUNK2, 128), jnp.float32),
            pltpu.VMEM((2, CHUNK2, D_MODEL), jnp.float32),
            pltpu.SemaphoreType.DMA,
            pltpu.SemaphoreType.DMA,
            pltpu.SemaphoreType.DMA,
        ],
        compiler_params=pltpu.CompilerParams(needs_layout_passes=False),
    )
    def body(idx_hbm, tab_hbm, out_hbm, idx_v, rows_v, obuf_v,
             isem, gsem, osem):
        wid = lax.axis_index("s") * NUM_CORES + lax.axis_index("c")
        row_base = wid * per_w
        irow_base = row_base // 128

        def idx_copy(g):
            off = pl.multiple_of(irow_base + g * NG, NG)
            return pltpu.make_async_copy(
                idx_hbm.at[pl.ds(off, NG)], idx_v.at[g % 4], isem)

        def gather_copy(g, b):
            return pltpu.make_async_copy(
                tab_hbm.at[idx_v.at[g % 4, b]],
                rows_v.at[g % 4, pl.ds(b * 128, 128)], gsem)

        def gather_wait(g, b):
            return pltpu.make_async_copy(
                tab_hbm.at[pl.ds(0, 128)],
                rows_v.at[g % 4, pl.ds(b * 128, 128)], gsem)

        def out_copy(g, buf):
            off = pl.multiple_of(row_base + g * CHUNK2, CHUNK2)
            return pltpu.make_async_copy(
                obuf_v.at[buf], out_hbm.at[pl.ds(off, CHUNK2)], osem)

        idx_copy(0).start()
        idx_copy(1).start()
        idx_copy(2).start()
        for j in range(2):
            idx_copy(j).wait()
            for b in range(NG):
                gather_copy(j, b).start()

        def step(g, carry):
            buf = g % 2

            @pl.when(g + 2 < steps)
            def _():
                idx_copy(g + 2).wait()
                for b in range(NG):
                    gather_copy(g + 2, b).start()

            for b in range(NG):
                gather_wait(g, b).wait()

            @pl.when(g + 3 < steps)
            def _():
                idx_copy(g + 3).start()

            def row_fix(i, c):
                q = [rows_v[g % 4, i, pl.ds(16 * k, 16)] for k in range(4)]
                s = (q[0] * q[0] + q[1] * q[1]) + (q[2] * q[2] + q[3] * q[3])
                nsv = _lane_allsum(s)
                scale = 2.0 / ((1.0 + EPS) - nsv)
                for k in range(4):
                    obuf_v[buf, i, pl.ds(16 * k, 16)] = q[k] * scale
                return c

            lax.fori_loop(0, CHUNK2, row_fix, 0, unroll=8)

            @pl.when(g >= 2)
            def _():
                out_copy(g - 2, buf).wait()
            out_copy(g, buf).start()
            return carry

        lax.fori_loop(0, steps, step, 0)
        out_copy(steps - 2, (steps - 2) % 2).wait()
        out_copy(steps - 1, (steps - 1) % 2).wait()

    return body(idx2d, pad_tab)


def kernel(token_ids, embeddings):
    bsz, seq = token_ids.shape
    vocab, d = embeddings.shape
    n_rows = bsz * seq
    n_full = (vocab // 128) * 128
    pad_tab = _repack_table(embeddings.T, embeddings[n_full:, :], vocab)
    idx2d = token_ids.reshape(n_rows // 128, 128).astype(jnp.int32)
    out = _gather_logmap(idx2d, pad_tab, n_rows)
    return out.reshape(bsz, seq, d)
